# baseline (speedup 1.0000x reference)
import jax
import jax.numpy as jnp
from jax import lax
from jax.experimental import pallas as pl
from jax.experimental.pallas import tpu as pltpu
from jax.experimental.pallas import tpu_sc as plsc

V = 2048
IN_CHANNELS = 256
K = 4
SCALE = 0.1
D, H, W = 32, 32, 32
N = D * H * W

NW = 32
VPW = N // NW
CSTRIDE = 129
SIMT_WORDS = 2176
SUB = 8
BIG = 1 << 30


def _bf16_rne(x):
    bits = lax.bitcast_convert_type(x, jnp.uint32)
    lsb = lax.shift_right_logical(bits, jnp.uint32(16)) & jnp.uint32(1)
    rounded = (bits + jnp.uint32(0x7FFF) + lsb) & jnp.uint32(0xFFFF0000)
    return lax.bitcast_convert_type(rounded, jnp.float32)


def _cart2sph(xyz):
    x = xyz[..., 0]
    y = xyz[..., 1]
    z = xyz[..., 2]
    rho = jnp.sqrt(x * x + y * y + z * z)
    rc = jnp.maximum(rho, 1e-12)
    theta = jnp.arccos(jnp.clip(z / rc, -1.0, 1.0))
    phi = jnp.arctan2(y, x)
    return rho, theta, phi


def _radial_unit(theta, phi):
    st = jnp.sin(theta)
    ct = jnp.cos(theta)
    cp = jnp.cos(phi)
    sp = jnp.sin(phi)
    return jnp.stack([st * cp, st * sp, ct], axis=-1)


def _proj_body(features_ref, wp_ref, b_ref, dsph_ref):
    dsph_ref[...] = jax.lax.dot_general(
        features_ref[...], wp_ref[...], (((1,), (0,)), ((), ())),
        preferred_element_type=jnp.float32,
    ) + b_ref[...]


def _gather16(vec, idx):
    return lax.gather(
        vec, idx.reshape(16, 1),
        lax.GatherDimensionNumbers(offset_dims=(), collapsed_slice_dims=(0,),
                                   start_index_map=(0,)),
        (1,), mode=lax.GatherScatterMode.PROMISE_IN_BOUNDS)


def _lane_splat(vec, lane):
    return _gather16(vec, jnp.full((16,), lane, jnp.int32))


def _sc_body(kx_h, ky_h, kz_h, dr_h, dt_h, dp_h,
             qxq_h, qyq_h, qzq_h, qx_h, qy_h, qz_h, rho_h, st_h,
             ox_h, oy_h, oz_h,
             kxv, kyv, kzv, drv, dtv, dpv,
             qxqv, qyqv, qzqv, qxv, qyv, qzv, rhov, stv,
             simt0, simt1, simt2, simt3, simt4, simt5, simt6, simt7,
             vb0, vb1, vb2, vb3, ib0, ib1, ib2, ib3, oxv, oyv, ozv):
    wid = lax.axis_index("s") * 2 + lax.axis_index("c")
    base = wid * VPW
    for src, dst in ((kx_h, kxv), (ky_h, kyv), (kz_h, kzv),
                     (dr_h, drv), (dt_h, dtv), (dp_h, dpv)):
        pltpu.sync_copy(src, dst)
    for src, dst in ((qxq_h, qxqv), (qyq_h, qyqv), (qzq_h, qzqv),
                     (qx_h, qxv), (qy_h, qyv), (qz_h, qzv),
                     (rho_h, rhov), (st_h, stv)):
        pltpu.sync_copy(src.at[pl.ds(base, VPW)], dst)

    iota16 = jax.lax.broadcasted_iota(jnp.int32, (16,), 0)
    scat_base = iota16 * CSTRIDE
    simts = (simt0, simt1, simt2, simt3, simt4, simt5, simt6, simt7)
    valbs = (vb0, vb1, vb2, vb3)
    idbs = (ib0, ib1, ib2, ib3)
    neg4 = jnp.full((16,), -4.0, jnp.float32)
    neg3 = jnp.full((16,), -3.0, jnp.float32)
    zi = jnp.zeros((16,), jnp.int32)

    def group_body(g, _):
        q16 = (qxqv[pl.ds(g * 16, 16)], qyqv[pl.ds(g * 16, 16)],
               qzqv[pl.ds(g * 16, 16)])
        for sub in range(16 // SUB):
            qs = [tuple(_lane_splat(qc, sub * SUB + j) for qc in q16)
                  for j in range(SUB)]

            init = tuple(neg4 for _ in range(SUB)) + tuple(
                zi for _ in range(SUB))

            @plsc.parallel_loop(0, V // 16, unroll=2, carry=init)
            def scan_carry(s, carry):
                ms = list(carry[:SUB])
                ss = list(carry[SUB:])
                kxc = kxv[pl.ds(s * 16, 16)]
                kyc = kyv[pl.ds(s * 16, 16)]
                kzc = kzv[pl.ds(s * 16, 16)]
                s_splat = jnp.full((16,), s, jnp.int32)
                for j in range(SUB):
                    qxs, qys, qzs = qs[j]
                    sim = qxs * kxc + (qys * kyc + qzs * kzc)
                    plsc.store_scatter(simts[j], [scat_base + s_splat], sim)
                    upd = sim > ms[j]
                    ms[j] = jnp.where(upd, sim, ms[j])
                    ss[j] = jnp.where(upd, s_splat, ss[j])
                return tuple(ms) + tuple(ss)

            ms = list(scan_carry[:SUB])
            sts = list(scan_carry[SUB:])
            vsels = [jnp.full((16,), g * 16 + sub * SUB + j, jnp.int32)
                     for j in range(SUB)]
            lane0 = iota16 == 0
            for r in range(K):
                gmaxs = [jnp.max(ms[j]) for j in range(SUB)]
                gids = [jnp.min(jnp.where(ms[j] == gmaxs[j],
                                          sts[j] * 16 + iota16, BIG))
                        for j in range(SUB)]
                for j in range(SUB):
                    plsc.store_scatter(valbs[r], [vsels[j]],
                                       jnp.full((16,), gmaxs[j]), mask=lane0)
                    plsc.store_scatter(idbs[r], [vsels[j]],
                                       jnp.full((16,), gids[j], jnp.int32),
                                       mask=lane0)
                if r < K - 1:
                    cols = [jnp.bitwise_and(gids[j], 15) * CSTRIDE
                            for j in range(SUB)]
                    s1s = [lax.shift_right_logical(gids[j], 4)
                           for j in range(SUB)]
                    for j in range(SUB):
                        plsc.store_scatter(
                            simts[j],
                            [jnp.full((16,), cols[j] + s1s[j], jnp.int32)],
                            neg3, mask=iota16 == 0)
                    colms = [neg4 for _ in range(SUB)]
                    colcs = [zi for _ in range(SUB)]
                    for c in range(V // 16 // 16):
                        csplat = jnp.full((16,), c, jnp.int32)
                        for j in range(SUB):
                            ch = plsc.load_gather(
                                simts[j],
                                [jnp.full((16,), cols[j] + c * 16, jnp.int32)
                                 + iota16])
                            updc = ch > colms[j]
                            colms[j] = jnp.where(updc, ch, colms[j])
                            colcs[j] = jnp.where(updc, csplat, colcs[j])
                    cmaxs = [jnp.max(colms[j]) for j in range(SUB)]
                    poss = [jnp.min(jnp.where(colms[j] == cmaxs[j],
                                              colcs[j] * 16 + iota16, BIG))
                            for j in range(SUB)]
                    for j in range(SUB):
                        lmask = scat_base == cols[j]
                        ms[j] = jnp.where(lmask, cmaxs[j], ms[j])
                        sts[j] = jnp.where(lmask, poss[j], sts[j])
        return 0

    lax.fori_loop(0, VPW // 16, group_body, 0)

    def comb_body(g, _):
        sl = pl.ds(g * 16, 16)
        wks = [1.0 / jnp.maximum(
                   (1.0 - jnp.minimum(valbs[r][sl], 1.0)) + 1e-8, 1e-8)
               for r in range(K)]
        wsum = ((wks[0] + wks[1]) + wks[2]) + wks[3]
        dcr = jnp.zeros((16,), jnp.float32)
        dct = jnp.zeros((16,), jnp.float32)
        dcp = jnp.zeros((16,), jnp.float32)
        for r in range(K):
            wn = wks[r] / wsum
            idx = idbs[r][sl]
            dcr = dcr + wn * plsc.load_gather(drv, [idx])
            dct = dct + wn * plsc.load_gather(dtv, [idx])
            dcp = dcp + wn * plsc.load_gather(dpv, [idx])
        qx = qxv[sl]
        qy = qyv[sl]
        qz = qzv[sl]
        rho = rhov[sl]
        st = stv[sl]
        rdt = rho * dct
        a = dcr + rdt * qz / st
        b = rho * dcp
        oxv[sl] = (a * qx - b * qy) * SCALE
        oyv[sl] = (a * qy + b * qx) * SCALE
        ozv[sl] = (dcr * qz - rdt * st) * SCALE
        return 0

    lax.fori_loop(0, VPW // 16, comb_body, 0)
    pltpu.sync_copy(oxv, ox_h.at[pl.ds(base, VPW)])
    pltpu.sync_copy(oyv, oy_h.at[pl.ds(base, VPW)])
    pltpu.sync_copy(ozv, oz_h.at[pl.ds(base, VPW)])


def kernel(features, vertex_pos_cartesian, out_size, Wp, b):
    del out_size
    features = features.reshape(V, IN_CHANNELS)
    verts = vertex_pos_cartesian.reshape(V, 3)
    b2 = b.reshape(1, 3)

    dsph = pl.pallas_call(
        _proj_body,
        out_shape=jax.ShapeDtypeStruct((V, 3), jnp.float32),
    )(features, Wp, b2)

    _, vt, vp = _cart2sph(verts)
    vu = _radial_unit(vt, vp)
    xs = jnp.arange(W, dtype=jnp.float32)
    ys = jnp.arange(H, dtype=jnp.float32)
    zs = jnp.arange(D, dtype=jnp.float32)
    gz, gy, gx = jnp.meshgrid(zs, ys, xs, indexing="ij")
    vox = jnp.stack([gx.reshape(-1), gy.reshape(-1), gz.reshape(-1)], axis=1)
    center = jnp.stack([(W - 1) * 0.5, (H - 1) * 0.5, (D - 1) * 0.5])
    vox = vox - center
    rho_v, th_v, ph_v = _cart2sph(vox)
    er = _radial_unit(th_v, ph_v)
    st_v = jnp.sin(th_v)
    erq = _bf16_rne(er)
    vuq = _bf16_rne(vu)

    f32 = jnp.float32
    sc = pl.kernel(
        _sc_body,
        out_type=[jax.ShapeDtypeStruct((N,), f32)] * 3,
        mesh=plsc.VectorSubcoreMesh(core_axis_name="c", subcore_axis_name="s"),
        compiler_params=pltpu.CompilerParams(needs_layout_passes=False),
        scratch_types=[pltpu.VMEM((V,), f32)] * 6
        + [pltpu.VMEM((VPW,), f32)] * 8
        + [pltpu.VMEM((SIMT_WORDS,), f32)] * 8
        + [pltpu.VMEM((VPW,), f32)] * 4
        + [pltpu.VMEM((VPW,), jnp.int32)] * 4
        + [pltpu.VMEM((VPW,), f32)] * 3,
    )
    ox, oy, oz = sc(vuq[:, 0], vuq[:, 1], vuq[:, 2],
                    dsph[:, 0], dsph[:, 1], dsph[:, 2],
                    erq[:, 0], erq[:, 1], erq[:, 2],
                    er[:, 0], er[:, 1], er[:, 2], rho_v, st_v)
    return jnp.stack([ox, oy, oz], axis=-1).reshape(D, H, W, 3)


# speedup vs baseline: 1.0560x; 1.0560x over previous
import jax
import jax.numpy as jnp
from jax import lax
from jax.experimental import pallas as pl
from jax.experimental.pallas import tpu as pltpu
from jax.experimental.pallas import tpu_sc as plsc

V = 2048
IN_CHANNELS = 256
K = 4
SCALE = 0.1
D, H, W = 32, 32, 32
N = D * H * W

NW = 32
VPW = N // NW
CSTRIDE = 129
SIMT_WORDS = 2176
SUB = 4
BIG = 1 << 30


def _bf16_rne(x):
    bits = lax.bitcast_convert_type(x, jnp.uint32)
    lsb = lax.shift_right_logical(bits, jnp.uint32(16)) & jnp.uint32(1)
    rounded = (bits + jnp.uint32(0x7FFF) + lsb) & jnp.uint32(0xFFFF0000)
    return lax.bitcast_convert_type(rounded, jnp.float32)


def _cart2sph(xyz):
    x = xyz[..., 0]
    y = xyz[..., 1]
    z = xyz[..., 2]
    rho = jnp.sqrt(x * x + y * y + z * z)
    rc = jnp.maximum(rho, 1e-12)
    theta = jnp.arccos(jnp.clip(z / rc, -1.0, 1.0))
    phi = jnp.arctan2(y, x)
    return rho, theta, phi


def _radial_unit(theta, phi):
    st = jnp.sin(theta)
    ct = jnp.cos(theta)
    cp = jnp.cos(phi)
    sp = jnp.sin(phi)
    return jnp.stack([st * cp, st * sp, ct], axis=-1)


def _proj_body(features_ref, wp_ref, b_ref, dsph_ref):
    dsph_ref[...] = jax.lax.dot_general(
        features_ref[...], wp_ref[...], (((1,), (0,)), ((), ())),
        preferred_element_type=jnp.float32,
    ) + b_ref[...]


def _gather16(vec, idx):
    return lax.gather(
        vec, idx.reshape(16, 1),
        lax.GatherDimensionNumbers(offset_dims=(), collapsed_slice_dims=(0,),
                                   start_index_map=(0,)),
        (1,), mode=lax.GatherScatterMode.PROMISE_IN_BOUNDS)


def _lane_splat(vec, lane):
    return _gather16(vec, jnp.full((16,), lane, jnp.int32))


def _sc_body(kx_h, ky_h, kz_h, dr_h, dt_h, dp_h,
             qxq_h, qyq_h, qzq_h, qx_h, qy_h, qz_h, rho_h, st_h,
             ox_h, oy_h, oz_h,
             kxv, kyv, kzv, drv, dtv, dpv,
             qxqv, qyqv, qzqv, qxv, qyv, qzv, rhov, stv,
             simt0, simt1, simt2, simt3,
             vb0, vb1, vb2, vb3, ib0, ib1, ib2, ib3, oxv, oyv, ozv):
    wid = lax.axis_index("s") * 2 + lax.axis_index("c")
    base = wid * VPW
    for src, dst in ((kx_h, kxv), (ky_h, kyv), (kz_h, kzv),
                     (dr_h, drv), (dt_h, dtv), (dp_h, dpv)):
        pltpu.sync_copy(src, dst)
    for src, dst in ((qxq_h, qxqv), (qyq_h, qyqv), (qzq_h, qzqv),
                     (qx_h, qxv), (qy_h, qyv), (qz_h, qzv),
                     (rho_h, rhov), (st_h, stv)):
        pltpu.sync_copy(src.at[pl.ds(base, VPW)], dst)

    iota16 = jax.lax.broadcasted_iota(jnp.int32, (16,), 0)
    scat_base = iota16 * CSTRIDE
    simts = (simt0, simt1, simt2, simt3)
    valbs = (vb0, vb1, vb2, vb3)
    idbs = (ib0, ib1, ib2, ib3)
    neg4 = jnp.full((16,), -4.0, jnp.float32)
    neg3 = jnp.full((16,), -3.0, jnp.float32)
    zi = jnp.zeros((16,), jnp.int32)

    def group_body(g, _):
        q16 = (qxqv[pl.ds(g * 16, 16)], qyqv[pl.ds(g * 16, 16)],
               qzqv[pl.ds(g * 16, 16)])
        for sub in range(16 // SUB):
            qs = [tuple(_lane_splat(qc, sub * SUB + j) for qc in q16)
                  for j in range(SUB)]

            init = tuple(neg4 for _ in range(SUB)) + tuple(
                zi for _ in range(SUB))

            @plsc.parallel_loop(0, V // 16, unroll=2, carry=init)
            def scan_carry(s, carry):
                ms = list(carry[:SUB])
                ss = list(carry[SUB:])
                kxc = kxv[pl.ds(s * 16, 16)]
                kyc = kyv[pl.ds(s * 16, 16)]
                kzc = kzv[pl.ds(s * 16, 16)]
                s_splat = jnp.full((16,), s, jnp.int32)
                for j in range(SUB):
                    qxs, qys, qzs = qs[j]
                    sim = qxs * kxc + (qys * kyc + qzs * kzc)
                    plsc.store_scatter(simts[j], [scat_base + s_splat], sim)
                    upd = sim > ms[j]
                    ms[j] = jnp.where(upd, sim, ms[j])
                    ss[j] = jnp.where(upd, s_splat, ss[j])
                return tuple(ms) + tuple(ss)

            ms = list(scan_carry[:SUB])
            sts = list(scan_carry[SUB:])
            vsels = [jnp.full((16,), g * 16 + sub * SUB + j, jnp.int32)
                     for j in range(SUB)]
            lane0 = iota16 == 0
            for r in range(K):
                gmaxs = [jnp.max(ms[j]) for j in range(SUB)]
                gids = [jnp.min(jnp.where(ms[j] == gmaxs[j],
                                          sts[j] * 16 + iota16, BIG))
                        for j in range(SUB)]
                for j in range(SUB):
                    plsc.store_scatter(valbs[r], [vsels[j]],
                                       jnp.full((16,), gmaxs[j]), mask=lane0)
                    plsc.store_scatter(idbs[r], [vsels[j]],
                                       jnp.full((16,), gids[j], jnp.int32),
                                       mask=lane0)
                if r < K - 1:
                    cols = [jnp.bitwise_and(gids[j], 15) * CSTRIDE
                            for j in range(SUB)]
                    s1s = [lax.shift_right_logical(gids[j], 4)
                           for j in range(SUB)]
                    for j in range(SUB):
                        plsc.store_scatter(
                            simts[j],
                            [jnp.full((16,), cols[j] + s1s[j], jnp.int32)],
                            neg3, mask=iota16 == 0)
                    colms = [neg4 for _ in range(SUB)]
                    colcs = [zi for _ in range(SUB)]
                    for c in range(V // 16 // 16):
                        csplat = jnp.full((16,), c, jnp.int32)
                        for j in range(SUB):
                            ch = plsc.load_gather(
                                simts[j],
                                [jnp.full((16,), cols[j] + c * 16, jnp.int32)
                                 + iota16])
                            updc = ch > colms[j]
                            colms[j] = jnp.where(updc, ch, colms[j])
                            colcs[j] = jnp.where(updc, csplat, colcs[j])
                    cmaxs = [jnp.max(colms[j]) for j in range(SUB)]
                    poss = [jnp.min(jnp.where(colms[j] == cmaxs[j],
                                              colcs[j] * 16 + iota16, BIG))
                            for j in range(SUB)]
                    for j in range(SUB):
                        lmask = scat_base == cols[j]
                        ms[j] = jnp.where(lmask, cmaxs[j], ms[j])
                        sts[j] = jnp.where(lmask, poss[j], sts[j])
        return 0

    lax.fori_loop(0, VPW // 16, group_body, 0)

    def comb_body(g, _):
        sl = pl.ds(g * 16, 16)
        wks = [1.0 / jnp.maximum(
                   (1.0 - jnp.minimum(valbs[r][sl], 1.0)) + 1e-8, 1e-8)
               for r in range(K)]
        wsum = ((wks[0] + wks[1]) + wks[2]) + wks[3]
        dcr = jnp.zeros((16,), jnp.float32)
        dct = jnp.zeros((16,), jnp.float32)
        dcp = jnp.zeros((16,), jnp.float32)
        for r in range(K):
            wn = wks[r] / wsum
            idx = idbs[r][sl]
            dcr = dcr + wn * plsc.load_gather(drv, [idx])
            dct = dct + wn * plsc.load_gather(dtv, [idx])
            dcp = dcp + wn * plsc.load_gather(dpv, [idx])
        qx = qxv[sl]
        qy = qyv[sl]
        qz = qzv[sl]
        rho = rhov[sl]
        st = stv[sl]
        rdt = rho * dct
        a = dcr + rdt * qz / st
        b = rho * dcp
        oxv[sl] = (a * qx - b * qy) * SCALE
        oyv[sl] = (a * qy + b * qx) * SCALE
        ozv[sl] = (dcr * qz - rdt * st) * SCALE
        return 0

    lax.fori_loop(0, VPW // 16, comb_body, 0)
    pltpu.sync_copy(oxv, ox_h.at[pl.ds(base, VPW)])
    pltpu.sync_copy(oyv, oy_h.at[pl.ds(base, VPW)])
    pltpu.sync_copy(ozv, oz_h.at[pl.ds(base, VPW)])


def kernel(features, vertex_pos_cartesian, out_size, Wp, b):
    del out_size
    features = features.reshape(V, IN_CHANNELS)
    verts = vertex_pos_cartesian.reshape(V, 3)
    b2 = b.reshape(1, 3)

    dsph = pl.pallas_call(
        _proj_body,
        out_shape=jax.ShapeDtypeStruct((V, 3), jnp.float32),
    )(features, Wp, b2)

    _, vt, vp = _cart2sph(verts)
    vu = _radial_unit(vt, vp)
    xs = jnp.arange(W, dtype=jnp.float32)
    ys = jnp.arange(H, dtype=jnp.float32)
    zs = jnp.arange(D, dtype=jnp.float32)
    gz, gy, gx = jnp.meshgrid(zs, ys, xs, indexing="ij")
    vox = jnp.stack([gx.reshape(-1), gy.reshape(-1), gz.reshape(-1)], axis=1)
    center = jnp.stack([(W - 1) * 0.5, (H - 1) * 0.5, (D - 1) * 0.5])
    vox = vox - center
    rho_v, th_v, ph_v = _cart2sph(vox)
    er = _radial_unit(th_v, ph_v)
    st_v = jnp.sin(th_v)
    erq = _bf16_rne(er)
    vuq = _bf16_rne(vu)

    f32 = jnp.float32
    sc = pl.kernel(
        _sc_body,
        out_type=[jax.ShapeDtypeStruct((N,), f32)] * 3,
        mesh=plsc.VectorSubcoreMesh(core_axis_name="c", subcore_axis_name="s"),
        compiler_params=pltpu.CompilerParams(needs_layout_passes=False),
        scratch_types=[pltpu.VMEM((V,), f32)] * 6
        + [pltpu.VMEM((VPW,), f32)] * 8
        + [pltpu.VMEM((SIMT_WORDS,), f32)] * 4
        + [pltpu.VMEM((VPW,), f32)] * 4
        + [pltpu.VMEM((VPW,), jnp.int32)] * 4
        + [pltpu.VMEM((VPW,), f32)] * 3,
    )
    ox, oy, oz = sc(vuq[:, 0], vuq[:, 1], vuq[:, 2],
                    dsph[:, 0], dsph[:, 1], dsph[:, 2],
                    erq[:, 0], erq[:, 1], erq[:, 2],
                    er[:, 0], er[:, 1], er[:, 2], rho_v, st_v)
    return jnp.stack([ox, oy, oz], axis=-1).reshape(D, H, W, 3)


# speedup vs baseline: 1.0734x; 1.0165x over previous
import jax
import jax.numpy as jnp
from jax import lax
from jax.experimental import pallas as pl
from jax.experimental.pallas import tpu as pltpu
from jax.experimental.pallas import tpu_sc as plsc

V = 2048
IN_CHANNELS = 256
K = 4
SCALE = 0.1
D, H, W = 32, 32, 32
N = D * H * W

NW = 32
VPW = N // NW
CSTRIDE = 129
SIMT_WORDS = 2176
SUB = 4
BIG = 1 << 30


def _bf16_rne(x):
    bits = lax.bitcast_convert_type(x, jnp.uint32)
    lsb = lax.shift_right_logical(bits, jnp.uint32(16)) & jnp.uint32(1)
    rounded = (bits + jnp.uint32(0x7FFF) + lsb) & jnp.uint32(0xFFFF0000)
    return lax.bitcast_convert_type(rounded, jnp.float32)


def _cart2sph(xyz):
    x = xyz[..., 0]
    y = xyz[..., 1]
    z = xyz[..., 2]
    rho = jnp.sqrt(x * x + y * y + z * z)
    rc = jnp.maximum(rho, 1e-12)
    theta = jnp.arccos(jnp.clip(z / rc, -1.0, 1.0))
    phi = jnp.arctan2(y, x)
    return rho, theta, phi


def _radial_unit(theta, phi):
    st = jnp.sin(theta)
    ct = jnp.cos(theta)
    cp = jnp.cos(phi)
    sp = jnp.sin(phi)
    return jnp.stack([st * cp, st * sp, ct], axis=-1)


def _proj_body(features_ref, wp_ref, b_ref, dsph_ref):
    dsph_ref[...] = jax.lax.dot_general(
        features_ref[...], wp_ref[...], (((1,), (0,)), ((), ())),
        preferred_element_type=jnp.float32,
    ) + b_ref[...]


def _gather16(vec, idx):
    return lax.gather(
        vec, idx.reshape(16, 1),
        lax.GatherDimensionNumbers(offset_dims=(), collapsed_slice_dims=(0,),
                                   start_index_map=(0,)),
        (1,), mode=lax.GatherScatterMode.PROMISE_IN_BOUNDS)


def _lane_splat(vec, lane):
    return _gather16(vec, jnp.full((16,), lane, jnp.int32))


def _sc_body(kx_h, ky_h, kz_h, dr_h, dt_h, dp_h,
             qxq_h, qyq_h, qzq_h, qx_h, qy_h, qz_h, rho_h, st_h,
             ox_h, oy_h, oz_h,
             kxv, kyv, kzv, drv, dtv, dpv,
             qxqv, qyqv, qzqv, qxv, qyv, qzv, rhov, stv,
             simt0, simt1, simt2, simt3,
             vb0, vb1, vb2, vb3, ib0, ib1, ib2, ib3, oxv, oyv, ozv, dsem):
    wid = lax.axis_index("s") * 2 + lax.axis_index("c")
    base = wid * VPW
    copies = [(src, dst) for src, dst in
              ((kx_h, kxv), (ky_h, kyv), (kz_h, kzv),
               (dr_h, drv), (dt_h, dtv), (dp_h, dpv))]
    copies += [(src.at[pl.ds(base, VPW)], dst) for src, dst in
               ((qxq_h, qxqv), (qyq_h, qyqv), (qzq_h, qzqv),
                (qx_h, qxv), (qy_h, qyv), (qz_h, qzv),
                (rho_h, rhov), (st_h, stv))]
    handles = [pltpu.async_copy(src, dst, dsem) for src, dst in copies]
    for h in handles:
        h.wait()

    iota16 = jax.lax.broadcasted_iota(jnp.int32, (16,), 0)
    scat_base = iota16 * CSTRIDE
    simts = (simt0, simt1, simt2, simt3)
    valbs = (vb0, vb1, vb2, vb3)
    idbs = (ib0, ib1, ib2, ib3)
    neg4 = jnp.full((16,), -4.0, jnp.float32)
    neg3 = jnp.full((16,), -3.0, jnp.float32)
    zi = jnp.zeros((16,), jnp.int32)

    def group_body(g, _):
        q16 = (qxqv[pl.ds(g * 16, 16)], qyqv[pl.ds(g * 16, 16)],
               qzqv[pl.ds(g * 16, 16)])
        for sub in range(16 // SUB):
            qs = [tuple(_lane_splat(qc, sub * SUB + j) for qc in q16)
                  for j in range(SUB)]

            init = tuple(neg4 for _ in range(SUB)) + tuple(
                zi for _ in range(SUB))

            @plsc.parallel_loop(0, V // 16, unroll=2, carry=init)
            def scan_carry(s, carry):
                ms = list(carry[:SUB])
                ss = list(carry[SUB:])
                kxc = kxv[pl.ds(s * 16, 16)]
                kyc = kyv[pl.ds(s * 16, 16)]
                kzc = kzv[pl.ds(s * 16, 16)]
                s_splat = jnp.full((16,), s, jnp.int32)
                for j in range(SUB):
                    qxs, qys, qzs = qs[j]
                    sim = qxs * kxc + (qys * kyc + qzs * kzc)
                    plsc.store_scatter(simts[j], [scat_base + s_splat], sim)
                    upd = sim > ms[j]
                    ms[j] = jnp.where(upd, sim, ms[j])
                    ss[j] = jnp.where(upd, s_splat, ss[j])
                return tuple(ms) + tuple(ss)

            ms = list(scan_carry[:SUB])
            sts = list(scan_carry[SUB:])
            vsels = [jnp.full((16,), g * 16 + sub * SUB + j, jnp.int32)
                     for j in range(SUB)]
            lane0 = iota16 == 0
            for r in range(K):
                gmaxs = [jnp.max(ms[j]) for j in range(SUB)]
                gids = [jnp.min(jnp.where(ms[j] == gmaxs[j],
                                          sts[j] * 16 + iota16, BIG))
                        for j in range(SUB)]
                for j in range(SUB):
                    plsc.store_scatter(valbs[r], [vsels[j]],
                                       jnp.full((16,), gmaxs[j]), mask=lane0)
                    plsc.store_scatter(idbs[r], [vsels[j]],
                                       jnp.full((16,), gids[j], jnp.int32),
                                       mask=lane0)
                if r < K - 1:
                    cols = [jnp.bitwise_and(gids[j], 15) * CSTRIDE
                            for j in range(SUB)]
                    s1s = [lax.shift_right_logical(gids[j], 4)
                           for j in range(SUB)]
                    for j in range(SUB):
                        plsc.store_scatter(
                            simts[j],
                            [jnp.full((16,), cols[j] + s1s[j], jnp.int32)],
                            neg3, mask=iota16 == 0)
                    colms = [neg4 for _ in range(SUB)]
                    colcs = [zi for _ in range(SUB)]
                    for c in range(V // 16 // 16):
                        csplat = jnp.full((16,), c, jnp.int32)
                        for j in range(SUB):
                            ch = plsc.load_gather(
                                simts[j],
                                [jnp.full((16,), cols[j] + c * 16, jnp.int32)
                                 + iota16])
                            updc = ch > colms[j]
                            colms[j] = jnp.where(updc, ch, colms[j])
                            colcs[j] = jnp.where(updc, csplat, colcs[j])
                    cmaxs = [jnp.max(colms[j]) for j in range(SUB)]
                    poss = [jnp.min(jnp.where(colms[j] == cmaxs[j],
                                              colcs[j] * 16 + iota16, BIG))
                            for j in range(SUB)]
                    for j in range(SUB):
                        lmask = scat_base == cols[j]
                        ms[j] = jnp.where(lmask, cmaxs[j], ms[j])
                        sts[j] = jnp.where(lmask, poss[j], sts[j])
        return 0

    lax.fori_loop(0, VPW // 16, group_body, 0)

    def comb_body(g, _):
        sl = pl.ds(g * 16, 16)
        wks = [1.0 / jnp.maximum(
                   (1.0 - jnp.minimum(valbs[r][sl], 1.0)) + 1e-8, 1e-8)
               for r in range(K)]
        wsum = ((wks[0] + wks[1]) + wks[2]) + wks[3]
        dcr = jnp.zeros((16,), jnp.float32)
        dct = jnp.zeros((16,), jnp.float32)
        dcp = jnp.zeros((16,), jnp.float32)
        for r in range(K):
            wn = wks[r] / wsum
            idx = idbs[r][sl]
            dcr = dcr + wn * plsc.load_gather(drv, [idx])
            dct = dct + wn * plsc.load_gather(dtv, [idx])
            dcp = dcp + wn * plsc.load_gather(dpv, [idx])
        qx = qxv[sl]
        qy = qyv[sl]
        qz = qzv[sl]
        rho = rhov[sl]
        st = stv[sl]
        rdt = rho * dct
        a = dcr + rdt * qz / st
        b = rho * dcp
        oxv[sl] = (a * qx - b * qy) * SCALE
        oyv[sl] = (a * qy + b * qx) * SCALE
        ozv[sl] = (dcr * qz - rdt * st) * SCALE
        return 0

    lax.fori_loop(0, VPW // 16, comb_body, 0)
    pltpu.sync_copy(oxv, ox_h.at[pl.ds(base, VPW)])
    pltpu.sync_copy(oyv, oy_h.at[pl.ds(base, VPW)])
    pltpu.sync_copy(ozv, oz_h.at[pl.ds(base, VPW)])


def kernel(features, vertex_pos_cartesian, out_size, Wp, b):
    del out_size
    features = features.reshape(V, IN_CHANNELS)
    verts = vertex_pos_cartesian.reshape(V, 3)
    b2 = b.reshape(1, 3)

    dsph = pl.pallas_call(
        _proj_body,
        out_shape=jax.ShapeDtypeStruct((V, 3), jnp.float32),
    )(features, Wp, b2)

    _, vt, vp = _cart2sph(verts)
    vu = _radial_unit(vt, vp)
    xs = jnp.arange(W, dtype=jnp.float32)
    ys = jnp.arange(H, dtype=jnp.float32)
    zs = jnp.arange(D, dtype=jnp.float32)
    gz, gy, gx = jnp.meshgrid(zs, ys, xs, indexing="ij")
    vox = jnp.stack([gx.reshape(-1), gy.reshape(-1), gz.reshape(-1)], axis=1)
    center = jnp.stack([(W - 1) * 0.5, (H - 1) * 0.5, (D - 1) * 0.5])
    vox = vox - center
    rho_v, th_v, ph_v = _cart2sph(vox)
    er = _radial_unit(th_v, ph_v)
    st_v = jnp.sin(th_v)
    erq = _bf16_rne(er)
    vuq = _bf16_rne(vu)

    f32 = jnp.float32
    sc = pl.kernel(
        _sc_body,
        out_type=[jax.ShapeDtypeStruct((N,), f32)] * 3,
        mesh=plsc.VectorSubcoreMesh(core_axis_name="c", subcore_axis_name="s"),
        compiler_params=pltpu.CompilerParams(needs_layout_passes=False),
        scratch_types=[pltpu.VMEM((V,), f32)] * 6
        + [pltpu.VMEM((VPW,), f32)] * 8
        + [pltpu.VMEM((SIMT_WORDS,), f32)] * 4
        + [pltpu.VMEM((VPW,), f32)] * 4
        + [pltpu.VMEM((VPW,), jnp.int32)] * 4
        + [pltpu.VMEM((VPW,), f32)] * 3
        + [pltpu.SemaphoreType.DMA],
    )
    ox, oy, oz = sc(vuq[:, 0], vuq[:, 1], vuq[:, 2],
                    dsph[:, 0], dsph[:, 1], dsph[:, 2],
                    erq[:, 0], erq[:, 1], erq[:, 2],
                    er[:, 0], er[:, 1], er[:, 2], rho_v, st_v)
    return jnp.stack([ox, oy, oz], axis=-1).reshape(D, H, W, 3)
